# Initial kernel scaffold; baseline (speedup 1.0000x reference)
import jax
import jax.numpy as jnp
from jax.experimental import pallas as pl


def kernel(image_features, text_features, image_ids, text_ids, epoch, max_epoch, s_I, s_T, b_I, b_T, tau_I, tau_T, u_I, u_T):
    raise NotImplementedError("write your pallas kernel here")


# speedup vs baseline: 1.1100x; 1.1100x over previous
import functools

import jax
import jax.numpy as jnp
from jax import lax
from jax.experimental import pallas as pl
from jax.experimental.pallas import tpu as pltpu
from jax.experimental.pallas import tpu_sc as plsc

N = 2900000
GAMMA = 0.8
RHO = 8.0
EPS = 1e-14
B, D = 256, 256

_NC, _NS = 2, 16
_CHUNKS = 4
_CW = B // _CHUNKS


def _gather_body(tau_i_hbm, b_i_hbm, s_i_hbm, tau_t_hbm, b_t_hbm, s_t_hbm,
                 img_ids_hbm, txt_ids_hbm,
                 o_tau_i, o_b_i, o_s_i, o_tau_t, o_b_t, o_s_t,
                 idx_v, val_v, sem):
    wid = lax.axis_index("s") * _NC + lax.axis_index("c")
    tid = wid // _CHUNKS
    chunk = wid % _CHUNKS
    base = pl.multiple_of(chunk * _CW, _CW)

    jobs = (
        (tau_i_hbm, img_ids_hbm, o_tau_i),
        (b_i_hbm, img_ids_hbm, o_b_i),
        (s_i_hbm, img_ids_hbm, o_s_i),
        (tau_t_hbm, txt_ids_hbm, o_tau_t),
        (b_t_hbm, txt_ids_hbm, o_b_t),
        (s_t_hbm, txt_ids_hbm, o_s_t),
    )
    for t, (tab, ids, out) in enumerate(jobs):
        @pl.when(tid == t)
        def _():
            pltpu.sync_copy(ids.at[pl.ds(base, _CW)], idx_v)
            pltpu.async_copy(tab.at[idx_v], val_v, sem).wait()
            pltpu.sync_copy(val_v, out.at[pl.ds(base, _CW)])


@functools.cache
def _make_gather_call():
    return functools.partial(
        pl.kernel,
        out_type=[jax.ShapeDtypeStruct((B,), jnp.float32)] * 6,
        mesh=plsc.VectorSubcoreMesh(core_axis_name="c", subcore_axis_name="s",
                                    num_cores=_NC, num_subcores=_NS),
        scratch_types=[
            pltpu.VMEM((_CW,), jnp.int32),
            pltpu.VMEM((_CW,), jnp.float32),
            pltpu.SemaphoreType.DMA,
        ],
    )(_gather_body)


def _dense_body(img_ref, txt_ref, tau_i_ref, tau_t_ref, b_i_ref, b_t_ref,
                s_i_ref, s_t_ref, isf_ref, loss_ref):
    sim = lax.dot_general(img_ref[...], txt_ref[...],
                          (((1,), (1,)), ((), ())),
                          preferred_element_type=jnp.float32)
    r = lax.broadcasted_iota(jnp.int32, (B, B), 0)
    c = lax.broadcasted_iota(jnp.int32, (B, B), 1)
    on_diag = r == c
    zero = jnp.zeros_like(sim)
    diag_col = jnp.sum(jnp.where(on_diag, sim, zero), axis=1, keepdims=True)
    diag_row = jnp.sum(jnp.where(on_diag, sim, zero), axis=0, keepdims=True)

    image_diffs = sim - diag_col
    text_diffs = sim - diag_row
    idt = image_diffs / tau_i_ref[...]
    tdt = text_diffs / tau_t_ref[...]

    b_i_new = jnp.maximum(jnp.max(idt, axis=1, keepdims=True), b_i_ref[...])
    b_t_new = jnp.maximum(jnp.max(tdt, axis=0, keepdims=True), b_t_ref[...])

    exp_i = jnp.where(on_diag, zero, jnp.exp(idt - b_i_new))
    exp_t = jnp.where(on_diag, zero, jnp.exp(tdt - b_t_new))

    g_i = jnp.sum(exp_i, axis=1, keepdims=True)
    g_t = jnp.sum(exp_t, axis=0, keepdims=True)

    isf = isf_ref[0, 0]
    s_i_upd = (1.0 - GAMMA) * s_i_ref[...] * jnp.exp(b_i_ref[...] - b_i_new) \
        + GAMMA * g_i
    s_t_upd = (1.0 - GAMMA) * s_t_ref[...] * jnp.exp(b_t_ref[...] - b_t_new) \
        + GAMMA * g_t
    s_i_v = isf * g_i + (1.0 - isf) * s_i_upd
    s_t_v = isf * g_t + (1.0 - isf) * s_t_upd

    w_i = exp_i / jnp.maximum(s_i_v, EPS)
    w_t = exp_t / jnp.maximum(s_t_v, EPS)

    loss = jnp.sum(w_i * image_diffs) / B + jnp.sum(w_t * text_diffs) / B
    loss_ref[0, 0] = loss


_dense_call = pl.pallas_call(
    _dense_body,
    out_shape=jax.ShapeDtypeStruct((1, 1), jnp.float32),
    out_specs=pl.BlockSpec(memory_space=pltpu.SMEM),
)


def kernel(image_features, text_features, image_ids, text_ids, epoch,
           max_epoch, s_I, s_T, b_I, b_T, tau_I, tau_T, u_I, u_T):
    tau_img, b_i_g, s_i_g, tau_txt, b_t_g, s_t_g = _make_gather_call()(
        tau_I, b_I, s_I, tau_T, b_T, s_T, image_ids, text_ids)

    isf = jnp.where(jnp.asarray(epoch) == 0, 1.0, 0.0).astype(
        jnp.float32).reshape(1, 1)

    loss = _dense_call(
        image_features, text_features,
        tau_img.reshape(B, 1), tau_txt.reshape(1, B),
        b_i_g.reshape(B, 1), b_t_g.reshape(1, B),
        s_i_g.reshape(B, 1), s_t_g.reshape(1, B),
        isf)

    return (loss.reshape(()), tau_img, tau_txt)


# speedup vs baseline: 1.3469x; 1.2134x over previous
import functools

import jax
import jax.numpy as jnp
from jax import lax
from jax.experimental import pallas as pl
from jax.experimental.pallas import tpu as pltpu
from jax.experimental.pallas import tpu_sc as plsc

N = 2900000
GAMMA = 0.8
RHO = 8.0
EPS = 1e-14
B, D = 256, 256

_NC, _NS = 2, 16
_CHUNKS = 4
_CW = B // _CHUNKS


def _gather_body(tau_i_hbm, b_i_hbm, s_i_hbm, tau_t_hbm, b_t_hbm, s_t_hbm,
                 img_ids_hbm, txt_ids_hbm,
                 blk, o_tau_img, o_tau_txt,
                 idx_v, val_v, sem):
    wid = lax.axis_index("s") * _NC + lax.axis_index("c")
    job = wid // _CHUNKS
    chunk = wid % _CHUNKS
    base = pl.multiple_of(chunk * _CW, _CW)

    jobs = (
        (tau_i_hbm, img_ids_hbm, 0),
        (b_i_hbm, img_ids_hbm, 1),
        (s_i_hbm, img_ids_hbm, 2),
        (tau_t_hbm, txt_ids_hbm, 3),
        (b_t_hbm, txt_ids_hbm, 4),
        (s_t_hbm, txt_ids_hbm, 5),
    )
    for t, (tab, ids, row) in enumerate(jobs):
        @pl.when(job == t)
        def _():
            pltpu.sync_copy(ids.at[pl.ds(base, _CW)], idx_v)
            pltpu.async_copy(tab.at[idx_v], val_v, sem).wait()
            pltpu.sync_copy(val_v, blk.at[row, pl.ds(base, _CW)])

    for t, (tab, ids, out) in enumerate(
            ((tau_i_hbm, img_ids_hbm, o_tau_img),
             (tau_t_hbm, txt_ids_hbm, o_tau_txt))):
        @pl.when(job == 6 + t)
        def _():
            pltpu.sync_copy(ids.at[pl.ds(base, _CW)], idx_v)
            pltpu.async_copy(tab.at[idx_v], val_v, sem).wait()
            pltpu.sync_copy(val_v, out.at[pl.ds(base, _CW)])


@functools.cache
def _make_gather_call():
    return functools.partial(
        pl.kernel,
        out_type=[
            jax.ShapeDtypeStruct((8, B), jnp.float32),
            jax.ShapeDtypeStruct((B,), jnp.float32),
            jax.ShapeDtypeStruct((B,), jnp.float32),
        ],
        mesh=plsc.VectorSubcoreMesh(core_axis_name="c", subcore_axis_name="s",
                                    num_cores=_NC, num_subcores=_NS),
        scratch_types=[
            pltpu.VMEM((_CW,), jnp.int32),
            pltpu.VMEM((_CW,), jnp.float32),
            pltpu.SemaphoreType.DMA,
        ],
    )(_gather_body)


def _dense_body(img_ref, txt_ref, blk_ref, epoch_ref, loss_ref):
    x = lax.dot_general(img_ref[...], txt_ref[...],
                        (((1,), (1,)), ((), ())),
                        preferred_element_type=jnp.float32)
    y = lax.dot_general(txt_ref[...], img_ref[...],
                        (((1,), (1,)), ((), ())),
                        preferred_element_type=jnp.float32)
    r = lax.broadcasted_iota(jnp.int32, (B, B), 0)
    c = lax.broadcasted_iota(jnp.int32, (B, B), 1)
    on_diag = r == c
    zero = jnp.zeros_like(x)
    diag_row = jnp.sum(jnp.where(on_diag, x, zero), axis=0, keepdims=True)

    isf = jnp.where(epoch_ref[0] == 0, 1.0, 0.0)

    def side(m, tau_row, b_row, s_row):
        diffs = m - diag_row
        dt = diffs / tau_row
        b_new = jnp.maximum(jnp.max(dt, axis=0, keepdims=True), b_row)
        e = jnp.where(on_diag, zero, jnp.exp(dt - b_new))
        g = jnp.sum(e, axis=0, keepdims=True)
        s_upd = (1.0 - GAMMA) * s_row * jnp.exp(b_row - b_new) + GAMMA * g
        s_v = isf * g + (1.0 - isf) * s_upd
        w = e / jnp.maximum(s_v, EPS)
        return jnp.sum(w * diffs)

    img_loss = side(y, blk_ref[0:1, :], blk_ref[1:2, :], blk_ref[2:3, :])
    txt_loss = side(x, blk_ref[3:4, :], blk_ref[4:5, :], blk_ref[5:6, :])
    loss_ref[0, 0] = img_loss / B + txt_loss / B


_dense_call = pl.pallas_call(
    _dense_body,
    in_specs=[
        pl.BlockSpec(),
        pl.BlockSpec(),
        pl.BlockSpec(),
        pl.BlockSpec(memory_space=pltpu.SMEM),
    ],
    out_shape=jax.ShapeDtypeStruct((1, 1), jnp.float32),
    out_specs=pl.BlockSpec(memory_space=pltpu.SMEM),
)


def kernel(image_features, text_features, image_ids, text_ids, epoch,
           max_epoch, s_I, s_T, b_I, b_T, tau_I, tau_T, u_I, u_T):
    blk, tau_img, tau_txt = _make_gather_call()(
        tau_I, b_I, s_I, tau_T, b_T, s_T, image_ids, text_ids)

    epoch_arr = jnp.asarray(epoch, jnp.int32).reshape(1)
    loss = _dense_call(image_features, text_features, blk, epoch_arr)

    return (loss.reshape(()), tau_img, tau_txt)


# speedup vs baseline: 1.3521x; 1.0038x over previous
import functools

import jax
import jax.numpy as jnp
from jax import lax
from jax.experimental import pallas as pl
from jax.experimental.pallas import tpu as pltpu
from jax.experimental.pallas import tpu_sc as plsc

N = 2900000
GAMMA = 0.8
RHO = 8.0
EPS = 1e-14
B, D = 256, 256

_NC, _NS = 2, 16
_NW = _NC * _NS
_CW = B // _NW
_NT = 6


def _gather_body(tau_i_hbm, b_i_hbm, s_i_hbm, tau_t_hbm, b_t_hbm, s_t_hbm,
                 img_ids_hbm, txt_ids_hbm, blk,
                 idx_v, val_v, sem):
    wid = lax.axis_index("s") * _NC + lax.axis_index("c")
    base = pl.multiple_of(wid * _CW, _CW)

    tabs = (tau_i_hbm, b_i_hbm, s_i_hbm, tau_t_hbm, b_t_hbm, s_t_hbm)
    ids = (img_ids_hbm, img_ids_hbm, img_ids_hbm,
           txt_ids_hbm, txt_ids_hbm, txt_ids_hbm)

    copies = [pltpu.async_copy(ids[t].at[pl.ds(base, _CW)], idx_v.at[t], sem)
              for t in range(_NT)]
    for cp in copies:
        cp.wait()
    copies = [pltpu.async_copy(tabs[t].at[idx_v.at[t]], val_v.at[t], sem)
              for t in range(_NT)]
    for cp in copies:
        cp.wait()
    copies = [pltpu.async_copy(val_v.at[t], blk.at[t, pl.ds(base, _CW)], sem)
              for t in range(_NT)]
    for cp in copies:
        cp.wait()


@functools.cache
def _make_gather_call():
    return functools.partial(
        pl.kernel,
        out_type=jax.ShapeDtypeStruct((8, B), jnp.float32),
        mesh=plsc.VectorSubcoreMesh(core_axis_name="c", subcore_axis_name="s",
                                    num_cores=_NC, num_subcores=_NS),
        scratch_types=[
            pltpu.VMEM((_NT, _CW), jnp.int32),
            pltpu.VMEM((_NT, _CW), jnp.float32),
            pltpu.SemaphoreType.DMA,
        ],
    )(_gather_body)


def _dense_body(img_ref, txt_ref, blk_ref, epoch_ref, loss_ref,
                tau_img_ref, tau_txt_ref):
    x = lax.dot_general(img_ref[...], txt_ref[...],
                        (((1,), (1,)), ((), ())),
                        preferred_element_type=jnp.float32)
    y = lax.dot_general(txt_ref[...], img_ref[...],
                        (((1,), (1,)), ((), ())),
                        preferred_element_type=jnp.float32)
    r = lax.broadcasted_iota(jnp.int32, (B, B), 0)
    c = lax.broadcasted_iota(jnp.int32, (B, B), 1)
    on_diag = r == c
    zero = jnp.zeros_like(x)
    diag_row = jnp.sum(jnp.where(on_diag, x, zero), axis=0, keepdims=True)

    isf = jnp.where(epoch_ref[0] == 0, 1.0, 0.0)

    def side(m, tau_row, b_row, s_row):
        diffs = m - diag_row
        dt = diffs / tau_row
        b_new = jnp.maximum(jnp.max(dt, axis=0, keepdims=True), b_row)
        e = jnp.where(on_diag, zero, jnp.exp(dt - b_new))
        g = jnp.sum(e, axis=0, keepdims=True)
        s_upd = (1.0 - GAMMA) * s_row * jnp.exp(b_row - b_new) + GAMMA * g
        s_v = isf * g + (1.0 - isf) * s_upd
        w = e / jnp.maximum(s_v, EPS)
        return jnp.sum(w * diffs)

    img_loss = side(y, blk_ref[0:1, :], blk_ref[1:2, :], blk_ref[2:3, :])
    txt_loss = side(x, blk_ref[3:4, :], blk_ref[4:5, :], blk_ref[5:6, :])
    loss_ref[0, 0] = img_loss / B + txt_loss / B
    tau_img_ref[...] = lax.squeeze(blk_ref[0:1, :], (0,))
    tau_txt_ref[...] = lax.squeeze(blk_ref[3:4, :], (0,))


_dense_call = pl.pallas_call(
    _dense_body,
    in_specs=[
        pl.BlockSpec(),
        pl.BlockSpec(),
        pl.BlockSpec(),
        pl.BlockSpec(memory_space=pltpu.SMEM),
    ],
    out_shape=[
        jax.ShapeDtypeStruct((1, 1), jnp.float32),
        jax.ShapeDtypeStruct((B,), jnp.float32),
        jax.ShapeDtypeStruct((B,), jnp.float32),
    ],
    out_specs=[
        pl.BlockSpec(memory_space=pltpu.SMEM),
        pl.BlockSpec(),
        pl.BlockSpec(),
    ],
)


def kernel(image_features, text_features, image_ids, text_ids, epoch,
           max_epoch, s_I, s_T, b_I, b_T, tau_I, tau_T, u_I, u_T):
    blk = _make_gather_call()(
        tau_I, b_I, s_I, tau_T, b_T, s_T, image_ids, text_ids)

    epoch_arr = jnp.asarray(epoch, jnp.int32).reshape(1)
    loss, tau_img, tau_txt = _dense_call(
        image_features, text_features, blk, epoch_arr)

    return (loss.reshape(()), tau_img, tau_txt)


# speedup vs baseline: 1.3647x; 1.0093x over previous
import functools

import jax
import jax.numpy as jnp
from jax import lax
from jax.experimental import pallas as pl
from jax.experimental.pallas import tpu as pltpu
from jax.experimental.pallas import tpu_sc as plsc

N = 2900000
GAMMA = 0.8
RHO = 8.0
EPS = 1e-14
B, D = 256, 256

_NC, _NS = 2, 16
_NW = _NC * _NS
_CW = B // _NW
_NT = 6


def _gather_body(tau_i_hbm, b_i_hbm, s_i_hbm, tau_t_hbm, b_t_hbm, s_t_hbm,
                 img_ids_hbm, txt_ids_hbm, blk,
                 idx_v, val_v, sem):
    wid = lax.axis_index("s") * _NC + lax.axis_index("c")
    base = pl.multiple_of(wid * _CW, _CW)

    tabs = (tau_i_hbm, b_i_hbm, s_i_hbm, tau_t_hbm, b_t_hbm, s_t_hbm)
    ids = (img_ids_hbm, img_ids_hbm, img_ids_hbm,
           txt_ids_hbm, txt_ids_hbm, txt_ids_hbm)

    copies = [pltpu.async_copy(ids[t].at[pl.ds(base, _CW)], idx_v.at[t], sem)
              for t in range(_NT)]
    for cp in copies:
        cp.wait()
    copies = [pltpu.async_copy(tabs[t].at[idx_v.at[t]], val_v.at[t], sem)
              for t in range(_NT)]
    for cp in copies:
        cp.wait()
    copies = [pltpu.async_copy(val_v.at[t], blk.at[t, pl.ds(base, _CW)], sem)
              for t in range(_NT)]
    for cp in copies:
        cp.wait()


@functools.cache
def _make_gather_call():
    return functools.partial(
        pl.kernel,
        out_type=jax.ShapeDtypeStruct((8, B), jnp.float32),
        mesh=plsc.VectorSubcoreMesh(core_axis_name="c", subcore_axis_name="s",
                                    num_cores=_NC, num_subcores=_NS),
        scratch_types=[
            pltpu.VMEM((_NT, _CW), jnp.int32),
            pltpu.VMEM((_NT, _CW), jnp.float32),
            pltpu.SemaphoreType.DMA,
        ],
    )(_gather_body)


def _matmul_body(img_ref, txt_ref, x_ref, y_ref):
    x_ref[...] = lax.dot_general(img_ref[...], txt_ref[...],
                                 (((1,), (1,)), ((), ())),
                                 preferred_element_type=jnp.float32)
    y_ref[...] = lax.dot_general(txt_ref[...], img_ref[...],
                                 (((1,), (1,)), ((), ())),
                                 preferred_element_type=jnp.float32)


_matmul_call = pl.pallas_call(
    _matmul_body,
    out_shape=[
        jax.ShapeDtypeStruct((B, B), jnp.float32),
        jax.ShapeDtypeStruct((B, B), jnp.float32),
    ],
)


def _dense_body(x_ref, y_ref, blk_ref, epoch_ref, loss_ref,
                tau_img_ref, tau_txt_ref):
    x = x_ref[...]
    y = y_ref[...]
    r = lax.broadcasted_iota(jnp.int32, (B, B), 0)
    c = lax.broadcasted_iota(jnp.int32, (B, B), 1)
    on_diag = r == c
    zero = jnp.zeros_like(x)
    diag_row = jnp.sum(jnp.where(on_diag, x, zero), axis=0, keepdims=True)

    isf = jnp.where(epoch_ref[0] == 0, 1.0, 0.0)

    def side(m, tau_row, b_row, s_row):
        diffs = m - diag_row
        dt = diffs / tau_row
        b_new = jnp.maximum(jnp.max(dt, axis=0, keepdims=True), b_row)
        e = jnp.where(on_diag, zero, jnp.exp(dt - b_new))
        g = jnp.sum(e, axis=0, keepdims=True)
        s_upd = (1.0 - GAMMA) * s_row * jnp.exp(b_row - b_new) + GAMMA * g
        s_v = isf * g + (1.0 - isf) * s_upd
        w = e / jnp.maximum(s_v, EPS)
        return jnp.sum(w * diffs)

    img_loss = side(y, blk_ref[0:1, :], blk_ref[1:2, :], blk_ref[2:3, :])
    txt_loss = side(x, blk_ref[3:4, :], blk_ref[4:5, :], blk_ref[5:6, :])
    loss_ref[0, 0] = img_loss / B + txt_loss / B
    tau_img_ref[...] = lax.squeeze(blk_ref[0:1, :], (0,))
    tau_txt_ref[...] = lax.squeeze(blk_ref[3:4, :], (0,))


_dense_call = pl.pallas_call(
    _dense_body,
    in_specs=[
        pl.BlockSpec(),
        pl.BlockSpec(),
        pl.BlockSpec(),
        pl.BlockSpec(memory_space=pltpu.SMEM),
    ],
    out_shape=[
        jax.ShapeDtypeStruct((1, 1), jnp.float32),
        jax.ShapeDtypeStruct((B,), jnp.float32),
        jax.ShapeDtypeStruct((B,), jnp.float32),
    ],
    out_specs=[
        pl.BlockSpec(memory_space=pltpu.SMEM),
        pl.BlockSpec(),
        pl.BlockSpec(),
    ],
)


def kernel(image_features, text_features, image_ids, text_ids, epoch,
           max_epoch, s_I, s_T, b_I, b_T, tau_I, tau_T, u_I, u_T):
    blk = _make_gather_call()(
        tau_I, b_I, s_I, tau_T, b_T, s_T, image_ids, text_ids)

    epoch_arr = jnp.asarray(epoch, jnp.int32).reshape(1)
    x, y = _matmul_call(image_features, text_features)
    loss, tau_img, tau_txt = _dense_call(x, y, blk, epoch_arr)

    return (loss.reshape(()), tau_img, tau_txt)


# speedup vs baseline: 1.3676x; 1.0022x over previous
import functools

import jax
import jax.numpy as jnp
from jax import lax
from jax.experimental import pallas as pl
from jax.experimental.pallas import tpu as pltpu
from jax.experimental.pallas import tpu_sc as plsc

N = 2900000
GAMMA = 0.8
RHO = 8.0
EPS = 1e-14
B, D = 256, 256

_NC, _NS = 2, 16
_NW = _NC * _NS
_CW = B // _NW
_NT = 6


def _gather_body(tau_i_hbm, b_i_hbm, s_i_hbm, tau_t_hbm, b_t_hbm, s_t_hbm,
                 img_ids_hbm, txt_ids_hbm, blk,
                 idx_v, val_v, sem):
    wid = lax.axis_index("s") * _NC + lax.axis_index("c")
    base = pl.multiple_of(wid * _CW, _CW)

    tabs = (tau_i_hbm, b_i_hbm, s_i_hbm, tau_t_hbm, b_t_hbm, s_t_hbm)

    copies = [pltpu.async_copy(ids.at[pl.ds(base, _CW)], idx_v.at[j], sem)
              for j, ids in enumerate((img_ids_hbm, txt_ids_hbm))]
    for cp in copies:
        cp.wait()
    copies = [pltpu.async_copy(tabs[t].at[idx_v.at[t // 3]], val_v.at[t], sem)
              for t in range(_NT)]
    for cp in copies:
        cp.wait()
    copies = [pltpu.async_copy(val_v.at[t], blk.at[t, pl.ds(base, _CW)], sem)
              for t in range(_NT)]
    for cp in copies:
        cp.wait()


@functools.cache
def _make_gather_call():
    return functools.partial(
        pl.kernel,
        out_type=jax.ShapeDtypeStruct((8, B), jnp.float32),
        mesh=plsc.VectorSubcoreMesh(core_axis_name="c", subcore_axis_name="s",
                                    num_cores=_NC, num_subcores=_NS),
        scratch_types=[
            pltpu.VMEM((2, _CW), jnp.int32),
            pltpu.VMEM((_NT, _CW), jnp.float32),
            pltpu.SemaphoreType.DMA,
        ],
    )(_gather_body)


def _matmul_body(img_ref, txt_ref, x_ref, y_ref):
    x_ref[...] = lax.dot_general(img_ref[...], txt_ref[...],
                                 (((1,), (1,)), ((), ())),
                                 preferred_element_type=jnp.float32)
    y_ref[...] = lax.dot_general(txt_ref[...], img_ref[...],
                                 (((1,), (1,)), ((), ())),
                                 preferred_element_type=jnp.float32)


_matmul_call = pl.pallas_call(
    _matmul_body,
    out_shape=[
        jax.ShapeDtypeStruct((B, B), jnp.float32),
        jax.ShapeDtypeStruct((B, B), jnp.float32),
    ],
)


def _dense_body(x_ref, y_ref, blk_ref, epoch_ref, loss_ref,
                tau_img_ref, tau_txt_ref):
    x = x_ref[...]
    y = y_ref[...]
    r = lax.broadcasted_iota(jnp.int32, (B, B), 0)
    c = lax.broadcasted_iota(jnp.int32, (B, B), 1)
    on_diag = r == c
    zero = jnp.zeros_like(x)
    diag_row = jnp.sum(jnp.where(on_diag, x, zero), axis=0, keepdims=True)

    isf = jnp.where(epoch_ref[0] == 0, 1.0, 0.0)

    def side(m, tau_row, b_row, s_row):
        diffs = m - diag_row
        dt = diffs / tau_row
        b_new = jnp.maximum(jnp.max(dt, axis=0, keepdims=True), b_row)
        e = jnp.where(on_diag, zero, jnp.exp(dt - b_new))
        g = jnp.sum(e, axis=0, keepdims=True)
        s_upd = (1.0 - GAMMA) * s_row * jnp.exp(b_row - b_new) + GAMMA * g
        s_v = isf * g + (1.0 - isf) * s_upd
        w = e / jnp.maximum(s_v, EPS)
        return jnp.sum(w * diffs)

    img_loss = side(y, blk_ref[0:1, :], blk_ref[1:2, :], blk_ref[2:3, :])
    txt_loss = side(x, blk_ref[3:4, :], blk_ref[4:5, :], blk_ref[5:6, :])
    loss_ref[0, 0] = img_loss / B + txt_loss / B
    tau_img_ref[...] = lax.squeeze(blk_ref[0:1, :], (0,))
    tau_txt_ref[...] = lax.squeeze(blk_ref[3:4, :], (0,))


_dense_call = pl.pallas_call(
    _dense_body,
    in_specs=[
        pl.BlockSpec(),
        pl.BlockSpec(),
        pl.BlockSpec(),
        pl.BlockSpec(memory_space=pltpu.SMEM),
    ],
    out_shape=[
        jax.ShapeDtypeStruct((1, 1), jnp.float32),
        jax.ShapeDtypeStruct((B,), jnp.float32),
        jax.ShapeDtypeStruct((B,), jnp.float32),
    ],
    out_specs=[
        pl.BlockSpec(memory_space=pltpu.SMEM),
        pl.BlockSpec(),
        pl.BlockSpec(),
    ],
)


def kernel(image_features, text_features, image_ids, text_ids, epoch,
           max_epoch, s_I, s_T, b_I, b_T, tau_I, tau_T, u_I, u_T):
    blk = _make_gather_call()(
        tau_I, b_I, s_I, tau_T, b_T, s_T, image_ids, text_ids)

    epoch_arr = jnp.asarray(epoch, jnp.int32).reshape(1)
    x, y = _matmul_call(image_features, text_features)
    loss, tau_img, tau_txt = _dense_call(x, y, blk, epoch_arr)

    return (loss.reshape(()), tau_img, tau_txt)
